# baseline (speedup 1.0000x reference)
import jax
import jax.numpy as jnp
import numpy as np
from jax.experimental import pallas as pl
from jax.experimental.pallas import tpu as pltpu

_BSZ = 16
_NV = 8
_NB_NODE = 64
_NB_EDGE = 16
_BLK = 16384

_HIGH = jax.lax.Precision.HIGHEST

_LOG2E = float(np.log2(np.e))
_A16 = (_NB_EDGE / 30.0) ** 2 * _LOG2E
_A64 = (_NB_NODE / 20.0) ** 2 * _LOG2E
_tabc = np.zeros((128, 8), np.float32)
_tabc[:_NB_NODE, 0] = np.linspace(0.0, 20.0, _NB_NODE) * np.sqrt(_A64)
_off = -1.0 + (np.arange(128) // _NB_EDGE) * (2.0 / (_NV - 1))
_tabc[:, 1] = 2.0 * _off * _A16
_tabc[:, 2] = 3.0 * _off * _off * _A16
_tabc[:, 3] = (np.arange(128) % _NB_EDGE) * (30.0 / (_NB_EDGE - 1)) \
    * np.sqrt(_A16)
_tabc[:, 4] = -1.0 + (np.arange(128) % _NV) * (2.0 / (_NV - 1))


def _fused_kernel(tabc_ref, coords4_ref, bids_ref,
                  nf_ref, vpos_ref, edge_ref, sums_ref):
    i = pl.program_id(0)
    nblk = pl.num_programs(0) - 1

    @pl.when(i == 0)
    def _phase0():
        n = coords4_ref.shape[1]
        brow = bids_ref[...]
        gcol = jax.lax.broadcasted_iota(
            jnp.int32, (_BSZ, n), 0).astype(jnp.float32)
        onehot_t = (gcol == brow).astype(jnp.float32)
        sums_ref[...] = jax.lax.dot_general(
            coords4_ref[...], onehot_t, (((1,), (1,)), ((), ())),
            preferred_element_type=jnp.float32, precision=_HIGH)

    @pl.when(i > 0)
    def _phase1():
        c4t = coords4_ref[:, pl.ds((i - 1) * _BLK, _BLK)]
        brow = bids_ref[:, pl.ds((i - 1) * _BLK, _BLK)]
        gcol = jax.lax.broadcasted_iota(
            jnp.int32, (_BSZ, _BLK), 0).astype(jnp.float32)
        onehot_t = (gcol == brow).astype(jnp.float32)

        sums = sums_ref[...]
        counts = jnp.maximum(sums[3:4, :], 1.0)
        cents = sums[0:3, :] / counts

        c2g = jnp.sum(cents * cents, axis=0, keepdims=True)
        csg = jnp.sum(cents, axis=0, keepdims=True)
        gtab = jnp.concatenate([cents, c2g, csg], axis=0)
        pg = jax.lax.dot_general(
            gtab, onehot_t, (((1,), (0,)), ((), ())),
            preferred_element_type=jnp.float32, precision=_HIGH)

        x = c4t[0:1, :]
        y = c4t[1:2, :]
        z = c4t[2:3, :]
        d2 = (x * x + y * y + z * z
              - 2.0 * (x * pg[0:1, :] + y * pg[1:2, :] + z * pg[2:3, :])
              + pg[3:4, :])
        d2 = jnp.maximum(d2, 0.0)
        s = (x + y + z) - pg[4:5, :]

        d2c64 = jnp.sqrt(d2 * _A64)
        z64 = d2c64 - tabc_ref[0:_NB_NODE, 0:1]
        nft = jnp.exp2(z64 * (tabc_ref[0:_NB_NODE, 0:1] - d2c64))
        nf_ref[...] = jax.lax.transpose(nft, (1, 0))

        d2a = d2 * _A16
        dist2 = d2a - s * tabc_ref[:, 1:2] + tabc_ref[:, 2:3]
        dist2 = jnp.maximum(dist2, 1e-20)
        dist = dist2 * jax.lax.rsqrt(dist2)
        zz = dist - tabc_ref[:, 3:4]
        erbf = jnp.exp2(zz * (tabc_ref[:, 3:4] - dist))

        part = jax.lax.dot_general(
            erbf.astype(jnp.bfloat16), onehot_t.astype(jnp.bfloat16),
            (((1,), (1,)), ((), ())),
            preferred_element_type=jnp.float32)

        @pl.when(i == 1)
        def _():
            edge_ref[...] = jnp.zeros_like(edge_ref)
            lane = jax.lax.broadcasted_iota(jnp.int32, (_BSZ, 128), 1)
            gid = jax.lax.broadcasted_iota(jnp.int32, (_BSZ, 128), 0)
            rep_t = (lane // _NV == gid).astype(jnp.float32)
            vpt = jax.lax.dot_general(
                cents, rep_t, (((1,), (0,)), ((), ())),
                preferred_element_type=jnp.float32, precision=_HIGH)
            vpt = vpt + jax.lax.transpose(tabc_ref[:, 4:5], (1, 0))
            vpos_ref[...] = jax.lax.transpose(vpt, (1, 0))

        edge_ref[...] += part

        @pl.when(i == nblk)
        def _():
            edge_ref[...] = edge_ref[...] / counts


def kernel(coords, batch_ids):
    n_real = coords.shape[0]
    nblk = n_real // _BLK
    tabc = jnp.asarray(_tabc)
    bids_row = batch_ids.astype(jnp.float32).reshape(1, n_real)
    coords4t = jnp.concatenate(
        [coords.T, jnp.ones((1, n_real), jnp.float32)], axis=0)

    node_feats, vpos, edge_t = pl.pallas_call(
        _fused_kernel,
        grid=(nblk + 1,),
        in_specs=[
            pl.BlockSpec((128, 8), lambda i: (0, 0)),
            pl.BlockSpec((4, 16384), lambda i: (0, 0)),
            pl.BlockSpec((1, 16384), lambda i: (0, 0)),
        ],
        out_specs=[
            pl.BlockSpec((_BLK, _NB_NODE),
                         lambda i: (jnp.maximum(i - 1, 0), 0)),
            pl.BlockSpec((_BSZ * _NV, 3), lambda i: (0, 0)),
            pl.BlockSpec((_NV * _NB_EDGE, _BSZ), lambda i: (0, 0)),
        ],
        out_shape=[
            jax.ShapeDtypeStruct((n_real, _NB_NODE), jnp.float32),
            jax.ShapeDtypeStruct((_BSZ * _NV, 3), jnp.float32),
            jax.ShapeDtypeStruct((_NV * _NB_EDGE, _BSZ), jnp.float32),
        ],
        scratch_shapes=[pltpu.VMEM((4, _BSZ), jnp.float32)],
    )(tabc, coords4t, bids_row)

    vbatch = jnp.repeat(jnp.arange(_BSZ), _NV)
    edge_agg = edge_t.reshape(_NV, _NB_EDGE, _BSZ).transpose(2, 0, 1) \
        .reshape(_BSZ * _NV, _NB_EDGE)
    return vbatch, vpos, node_feats, edge_agg


# speedup vs baseline: 1.0364x; 1.0364x over previous
import jax
import jax.numpy as jnp
import numpy as np
from jax.experimental import pallas as pl
from jax.experimental.pallas import tpu as pltpu

_BSZ = 16
_NV = 8
_NB_NODE = 64
_NB_EDGE = 16
_BLK = 4096

_HIGH = jax.lax.Precision.HIGHEST

_LOG2E = float(np.log2(np.e))
_A16 = (_NB_EDGE / 30.0) ** 2 * _LOG2E
_A64 = (_NB_NODE / 20.0) ** 2 * _LOG2E
_tabc = np.zeros((128, 8), np.float32)
_tabc[:_NB_NODE, 0] = np.linspace(0.0, 20.0, _NB_NODE) * np.sqrt(_A64)
_off = -1.0 + (np.arange(128) // _NB_EDGE) * (2.0 / (_NV - 1))
_tabc[:, 1] = 2.0 * _off * _A16
_tabc[:, 2] = 3.0 * _off * _off * _A16
_tabc[:, 3] = (np.arange(128) % _NB_EDGE) * (30.0 / (_NB_EDGE - 1)) \
    * np.sqrt(_A16)
_tabc[:, 4] = -1.0 + (np.arange(128) % _NV) * (2.0 / (_NV - 1))


def _fused_kernel(tabc_ref, coords4_ref, bids_ref,
                  nf_ref, vpos_ref, edge_ref, sums_ref):
    i = pl.program_id(0)
    nblk = pl.num_programs(0) - 1

    @pl.when(i == 0)
    def _phase0():
        n = coords4_ref.shape[1]
        brow = bids_ref[...]
        gcol = jax.lax.broadcasted_iota(
            jnp.int32, (_BSZ, n), 0).astype(jnp.float32)
        onehot_t = (gcol == brow).astype(jnp.float32)
        sums_ref[...] = jax.lax.dot_general(
            coords4_ref[...], onehot_t, (((1,), (1,)), ((), ())),
            preferred_element_type=jnp.float32, precision=_HIGH)

    @pl.when(i > 0)
    def _phase1():
        c4t = coords4_ref[:, pl.ds((i - 1) * _BLK, _BLK)]
        brow = bids_ref[:, pl.ds((i - 1) * _BLK, _BLK)]
        gcol = jax.lax.broadcasted_iota(
            jnp.int32, (_BSZ, _BLK), 0).astype(jnp.float32)
        onehot_t = (gcol == brow).astype(jnp.float32)

        sums = sums_ref[...]
        counts = jnp.maximum(sums[3:4, :], 1.0)
        cents = sums[0:3, :] / counts

        c2g = jnp.sum(cents * cents, axis=0, keepdims=True)
        csg = jnp.sum(cents, axis=0, keepdims=True)
        gtab = jnp.concatenate([cents, c2g, csg], axis=0)
        pg = jax.lax.dot_general(
            gtab, onehot_t, (((1,), (0,)), ((), ())),
            preferred_element_type=jnp.float32, precision=_HIGH)

        x = c4t[0:1, :]
        y = c4t[1:2, :]
        z = c4t[2:3, :]
        d2 = (x * x + y * y + z * z
              - 2.0 * (x * pg[0:1, :] + y * pg[1:2, :] + z * pg[2:3, :])
              + pg[3:4, :])
        d2 = jnp.maximum(d2, 0.0)
        s = (x + y + z) - pg[4:5, :]

        d2c64 = jnp.sqrt(d2 * _A64)
        z64 = d2c64 - tabc_ref[0:_NB_NODE, 0:1]
        nft = jnp.exp2(z64 * (tabc_ref[0:_NB_NODE, 0:1] - d2c64))
        nf_ref[...] = jax.lax.transpose(nft, (1, 0))

        d2a = d2 * _A16
        dist2 = d2a - s * tabc_ref[:, 1:2] + tabc_ref[:, 2:3]
        dist2 = jnp.maximum(dist2, 1e-20)
        dist = dist2 * jax.lax.rsqrt(dist2)
        zz = dist - tabc_ref[:, 3:4]
        erbf = jnp.exp2(zz * (tabc_ref[:, 3:4] - dist))

        part = jax.lax.dot_general(
            erbf.astype(jnp.bfloat16), onehot_t.astype(jnp.bfloat16),
            (((1,), (1,)), ((), ())),
            preferred_element_type=jnp.float32)

        @pl.when(i == 1)
        def _():
            edge_ref[...] = jnp.zeros_like(edge_ref)
            lane = jax.lax.broadcasted_iota(jnp.int32, (_BSZ, 128), 1)
            gid = jax.lax.broadcasted_iota(jnp.int32, (_BSZ, 128), 0)
            rep_t = (lane // _NV == gid).astype(jnp.float32)
            vpt = jax.lax.dot_general(
                cents, rep_t, (((1,), (0,)), ((), ())),
                preferred_element_type=jnp.float32, precision=_HIGH)
            vpt = vpt + jax.lax.transpose(tabc_ref[:, 4:5], (1, 0))
            vpos_ref[...] = jax.lax.transpose(vpt, (1, 0))

        edge_ref[...] += part

        @pl.when(i == nblk)
        def _():
            edge_ref[...] = edge_ref[...] / counts


def kernel(coords, batch_ids):
    n_real = coords.shape[0]
    nblk = n_real // _BLK
    tabc = jnp.asarray(_tabc)
    bids_row = batch_ids.astype(jnp.float32).reshape(1, n_real)
    coords4t = jnp.concatenate(
        [coords.T, jnp.ones((1, n_real), jnp.float32)], axis=0)

    node_feats, vpos, edge_t = pl.pallas_call(
        _fused_kernel,
        grid=(nblk + 1,),
        in_specs=[
            pl.BlockSpec((128, 8), lambda i: (0, 0)),
            pl.BlockSpec((4, 16384), lambda i: (0, 0)),
            pl.BlockSpec((1, 16384), lambda i: (0, 0)),
        ],
        out_specs=[
            pl.BlockSpec((_BLK, _NB_NODE),
                         lambda i: (jnp.maximum(i - 1, 0), 0)),
            pl.BlockSpec((_BSZ * _NV, 3), lambda i: (0, 0)),
            pl.BlockSpec((_NV * _NB_EDGE, _BSZ), lambda i: (0, 0)),
        ],
        out_shape=[
            jax.ShapeDtypeStruct((n_real, _NB_NODE), jnp.float32),
            jax.ShapeDtypeStruct((_BSZ * _NV, 3), jnp.float32),
            jax.ShapeDtypeStruct((_NV * _NB_EDGE, _BSZ), jnp.float32),
        ],
        scratch_shapes=[pltpu.VMEM((4, _BSZ), jnp.float32)],
    )(tabc, coords4t, bids_row)

    vbatch = jnp.repeat(jnp.arange(_BSZ), _NV)
    edge_agg = edge_t.reshape(_NV, _NB_EDGE, _BSZ).transpose(2, 0, 1) \
        .reshape(_BSZ * _NV, _NB_EDGE)
    return vbatch, vpos, node_feats, edge_agg
